# baseline (speedup 1.0000x reference)
import functools
import jax
import jax.numpy as jnp
from jax import lax
from jax.experimental import pallas as pl
from jax.experimental.pallas import tpu as pltpu
from jax.experimental.pallas import tpu_sc as plsc

_N = 10000
_E = 320000
_F = 128
_FH = _F // 2
_G = 128

_NC = 2
_NS = 16
_NW = _NC * _NS
_CH = 80
_EPT = _E // _NS
_NCH = 252
_EPTP = _NCH * _CH
_TRASH = _N
_NTR = 512
_NA = _N + _NTR
_EG = 28
_NEG = _NCH // _EG
_CHD = 80
_EPW = _E // _NW
_NCHD = _EPW // _CHD
_TPR = 624
_WCH = 104
_NWC = _TPR // _WCH
_TAIL = _N - _NS * _TPR
_BLK = 1000
_NBLK = _N // _BLK


def _mesh():
    return plsc.VectorSubcoreMesh(core_axis_name="c", subcore_axis_name="s")


_SC_PARAMS = pltpu.CompilerParams(needs_layout_passes=False,
                                  use_tc_tiling_on_sc=False)


_DW = 16


@functools.partial(
    pl.kernel,
    mesh=_mesh(),
    compiler_params=_SC_PARAMS,
    out_type=[jax.ShapeDtypeStruct((_NC, _N, _DW), jnp.float32)],
    scratch_types=[
        pltpu.VMEM((_NCHD, _CHD), jnp.int32),
        pltpu.VMEM((_CHD, _DW), jnp.float32),
        pltpu.VMEM((_WCH, _DW), jnp.float32),
        pltpu.SemaphoreType.DMA,
        pltpu.VMEM_SHARED((_N, _DW), jnp.float32),
    ],
)
def _sc_deg(dst3_hbm, z1_hbm, ones_hbm, deg_out, dst2_v, ones_v, zst_v, sem,
            acc):
    c = lax.axis_index("c")
    s = lax.axis_index("s")
    w = c * _NS + s
    pltpu.sync_copy(z1_hbm, zst_v)
    pltpu.sync_copy(ones_hbm, ones_v)
    pltpu.sync_copy(dst3_hbm.at[w], dst2_v)
    for j in range(_NWC):
        pltpu.sync_copy(zst_v, acc.at[pl.ds(s * _TPR + j * _WCH, _WCH)])

    @pl.when(s == 0)
    def _():
        pltpu.sync_copy(zst_v.at[pl.ds(0, _TAIL)],
                        acc.at[pl.ds(_NS * _TPR, _TAIL)])

    plsc.subcore_barrier()

    def fire(i, carry):
        pltpu.async_copy(ones_v, acc.at[dst2_v.at[i]], sem, add=True)
        return carry

    lax.fori_loop(0, _NCHD, fire, 0)

    def drain(i, carry):
        pltpu.make_async_copy(ones_v, acc.at[dst2_v.at[i]], sem).wait()
        return carry

    lax.fori_loop(0, _NCHD, drain, 0)
    plsc.subcore_barrier()
    for j in range(_NWC):
        r0 = s * _TPR + j * _WCH
        pltpu.sync_copy(acc.at[pl.ds(r0, _WCH)], zst_v)
        pltpu.sync_copy(zst_v, deg_out.at[c, pl.ds(r0, _WCH)])

    @pl.when(s == 0)
    def _():
        pltpu.sync_copy(acc.at[pl.ds(_NS * _TPR, _TAIL)],
                        zst_v.at[pl.ds(0, _TAIL)])
        pltpu.sync_copy(zst_v.at[pl.ds(0, _TAIL)],
                        deg_out.at[c, pl.ds(_NS * _TPR, _TAIL)])


@functools.partial(
    pl.kernel,
    mesh=_mesh(),
    compiler_params=_SC_PARAMS,
    out_type=[jax.ShapeDtypeStruct((_NC, _N, _FH), jnp.float32),
              jax.ShapeDtypeStruct((_NS, _NCH, _CH), jnp.float32)],
    scratch_types=[
        pltpu.VMEM((_NCH, _CH), jnp.int32),
        pltpu.VMEM((_NCH, _CH), jnp.int32),
        pltpu.VMEM((_CH, _FH), jnp.float32),
        pltpu.VMEM((_CH, _FH), jnp.float32),
        pltpu.VMEM((_WCH, _FH), jnp.float32),
        pltpu.SemaphoreType.DMA,
        pltpu.SemaphoreType.DMA,
        pltpu.VMEM((_NA,), jnp.float32),
        pltpu.VMEM((_NCH, _CH), jnp.float32),
        pltpu.VMEM_SHARED((_NA, _FH), jnp.float32),
    ],
)
def _sc_agg(src3, dst3, x_hbm, z_hbm, svec_hbm, out_hbm, eout_hbm,
            src2, dst2, rb0, rb1, stage, sg0, sg1, svec_v, ebuf, acc):
    c = lax.axis_index("c")
    s = lax.axis_index("s")
    rows = [rb0, rb1]
    gsem = [sg0, sg1]

    pltpu.sync_copy(src3.at[s], src2)
    pltpu.sync_copy(dst3.at[s], dst2)
    xh = x_hbm.at[c]
    pltpu.async_copy(xh.at[src2.at[0]], rows[0], gsem[0])
    pltpu.async_copy(xh.at[src2.at[1]], rows[1], gsem[1])

    pltpu.sync_copy(z_hbm, stage)
    for j in range(_NWC):
        pltpu.sync_copy(stage, acc.at[pl.ds(s * _TPR + j * _WCH, _WCH)])

    @pl.when(s == 0)
    def _():
        pltpu.sync_copy(stage.at[pl.ds(0, _TAIL)],
                        acc.at[pl.ds(_NS * _TPR, _TAIL)])

    @pl.when(c == 0)
    def _():
        pltpu.sync_copy(svec_hbm, svec_v.at[pl.ds(0, _N)])

    plsc.subcore_barrier()

    def edge_scalars(ci, er):
        for j2 in range(_CH // 16):
            si = src2[ci, pl.ds(j2 * 16, 16)]
            di = dst2[ci, pl.ds(j2 * 16, 16)]
            gs = plsc.load_gather(svec_v, [si])
            gd = plsc.load_gather(svec_v, [di])
            ebuf[er, pl.ds(j2 * 16, 16)] = gs * gd

    def pair(k, carry):
        for j in range(2):
            ci = 2 * k + j

            @pl.when(c == 0)
            def _():
                edge_scalars(ci, ci)

            pltpu.make_async_copy(xh.at[src2.at[ci]], rows[j],
                                  gsem[j]).wait()
            pltpu.sync_copy(rows[j], acc.at[dst2.at[ci]], add=True)

            @pl.when(ci + 2 < _NCH)
            def _():
                pltpu.async_copy(xh.at[src2.at[ci + 2]], rows[j], gsem[j])
        return carry

    lax.fori_loop(0, _NCH // 2, pair, 0)

    @pl.when(c == 0)
    def _():
        pltpu.sync_copy(ebuf, eout_hbm.at[s])

    plsc.subcore_barrier()
    for j in range(_NWC):
        r0 = s * _TPR + j * _WCH
        pltpu.sync_copy(acc.at[pl.ds(r0, _WCH)], stage)
        pltpu.sync_copy(stage, out_hbm.at[c, pl.ds(r0, _WCH)])

    @pl.when(s == 0)
    def _():
        pltpu.sync_copy(acc.at[pl.ds(_NS * _TPR, _TAIL)],
                        stage.at[pl.ds(0, _TAIL)])
        pltpu.sync_copy(stage.at[pl.ds(0, _TAIL)],
                        out_hbm.at[c, pl.ds(_NS * _TPR, _TAIL)])


def _row_spec(width):
    return pl.BlockSpec((_BLK, width), lambda i: (i, 0))


def _split_spec():
    return pl.BlockSpec((_NC, _BLK, _FH), lambda i: (0, i, 0))


def _full_spec(shape):
    nd = len(shape)
    return pl.BlockSpec(shape, lambda i: (0,) * nd)


def _split_store(ref, y):
    ref[0] = y[:, :_FH]
    ref[1] = y[:, _FH:]


def _split_load(ref):
    return jnp.concatenate([ref[0], ref[1]], axis=-1)


def _tc_scale_body(da_ref, db_ref, x_ref, dis_ref, xs_ref):
    deg = da_ref[...] + db_ref[...]
    dis = lax.rsqrt(jnp.maximum(deg, 1.0))
    dis_ref[...] = dis
    _split_store(xs_ref, dis * x_ref[...])


def _tc_scale(dega, degb, x):
    return pl.pallas_call(
        _tc_scale_body,
        grid=(_NBLK,),
        in_specs=[_row_spec(1), _row_spec(1), _row_spec(_F)],
        out_specs=[_row_spec(1), _split_spec()],
        out_shape=[jax.ShapeDtypeStruct((_N, 1), jnp.float32),
                   jax.ShapeDtypeStruct((_NC, _N, _FH), jnp.float32)],
    )(dega, degb, x)


def _tc_layer_body(p_ref, s_ref, w_ref, y_ref):
    p = _split_load(p_ref)
    sv = s_ref[...]
    h = jnp.maximum(jnp.dot(sv * p, w_ref[...],
                            preferred_element_type=jnp.float32), 0.0)
    _split_store(y_ref, sv * h)


def _tc_layer(part, sv, w):
    return pl.pallas_call(
        _tc_layer_body,
        grid=(_NBLK,),
        in_specs=[_split_spec(), _row_spec(1), _full_spec((_F, _F))],
        out_specs=_split_spec(),
        out_shape=jax.ShapeDtypeStruct((_NC, _N, _FH), jnp.float32),
    )(part, sv, w)


def _tc_att_body(p_ref, dis_ref, w2_ref, wa_ref, ba_ref, nz_ref, tr_ref,
                 x_ref, att_ref, q_ref, xq_ref, info_ref, acc_ref):
    i = pl.program_id(0)
    p = _split_load(p_ref)
    dis = dis_ref[...]
    emb = jnp.maximum(jnp.dot(dis * p, w2_ref[...],
                              preferred_element_type=jnp.float32), 0.0)
    logits = jnp.dot(emb, wa_ref[...],
                     preferred_element_type=jnp.float32) + ba_ref[...]
    att = jax.nn.sigmoid(logits + jnp.where(tr_ref[...] != 0.0,
                                            nz_ref[...], 0.0))
    att_ref[...] = att
    q = dis * att
    q_ref[...] = q
    _split_store(xq_ref, q * x_ref[...])
    r = 0.7
    f = (att * jnp.log(att / r + 1e-6)
         + (1.0 - att) * jnp.log((1.0 - att) / (1.0 - r + 1e-6) + 1e-6))
    part = jnp.sum(f).reshape(1, 1)
    acc_ref[...] = jnp.where(i == 0, part, acc_ref[...] + part)

    @pl.when(i == _NBLK - 1)
    def _():
        info_ref[...] = acc_ref[...] / float(_N)


def _tc_att(part, dis, w2, wa, ba, noise, tr, x):
    return pl.pallas_call(
        _tc_att_body,
        grid=(_NBLK,),
        in_specs=[_split_spec(), _row_spec(1), _full_spec((_F, _F)),
                  _full_spec((_F, 1)), _full_spec((1, 1)), _row_spec(1),
                  _full_spec((1, 1)), _row_spec(_F)],
        out_specs=[_row_spec(1), _row_spec(1), _split_spec(),
                   _full_spec((1, 1))],
        out_shape=[jax.ShapeDtypeStruct((_N, 1), jnp.float32),
                   jax.ShapeDtypeStruct((_N, 1), jnp.float32),
                   jax.ShapeDtypeStruct((_NC, _N, _FH), jnp.float32),
                   jax.ShapeDtypeStruct((1, 1), jnp.float32)],
        scratch_shapes=[pltpu.VMEM((1, 1), jnp.float32)],
    )(part, dis, w2, wa, ba, noise, tr, x)


def _tc_pool_body(p_ref, q_ref, w4_ref, b_ref, sp_ref, pool_ref, cnt_ref):
    i = pl.program_id(0)
    p = _split_load(p_ref)
    s2 = jnp.maximum(q_ref[...] * jnp.dot(p, w4_ref[...],
                                          preferred_element_type=jnp.float32),
                     0.0)
    b = b_ref[...]
    iota = lax.broadcasted_iota(jnp.int32, (_BLK, _G), 1)
    m = (b == iota).astype(jnp.float32)
    dims = (((0,), (0,)), ((), ()))
    pool_d = lax.dot_general(m, s2, dims, preferred_element_type=jnp.float32)
    cnt_d = lax.dot_general(m, jnp.ones((_BLK, 1), jnp.float32), dims,
                            preferred_element_type=jnp.float32)

    @pl.when(i == 0)
    def _():
        pool_ref[...] = pool_d
        cnt_ref[...] = cnt_d

    @pl.when(i > 0)
    def _():
        pool_ref[...] += pool_d
        cnt_ref[...] += cnt_d

    @pl.when(i == _NBLK - 1)
    def _():
        sp_ref[...] = pool_ref[...] / jnp.maximum(cnt_ref[...], 1.0)


def _tc_pool(part, q, w4, batch2d):
    return pl.pallas_call(
        _tc_pool_body,
        grid=(_NBLK,),
        in_specs=[_split_spec(), _row_spec(1), _full_spec((_F, _F)),
                  _row_spec(1)],
        out_specs=_full_spec((_G, _F)),
        out_shape=jax.ShapeDtypeStruct((_G, _F), jnp.float32),
        scratch_shapes=[pltpu.VMEM((_G, _F), jnp.float32),
                        pltpu.VMEM((_G, 1), jnp.float32)],
    )(part, q, w4, batch2d)


def kernel(edge_index, inputs, epoch, training, batch, W1, W2, Wa, ba, W3, W4):
    pad = _EPTP - _EPT
    src3 = jnp.pad(edge_index[0].reshape(_NS, _EPT),
                   ((0, 0), (0, pad))).reshape(_NS, _NCH, _CH)
    padv = _TRASH + (jnp.arange(pad, dtype=jnp.int32) % _NTR)
    dst3 = jnp.concatenate(
        [edge_index[1].reshape(_NS, _EPT),
         jnp.broadcast_to(padv, (_NS, pad))], axis=1).reshape(_NS, _NCH, _CH)
    dst3d = edge_index[1].reshape(_NW, _NCHD, _CHD)
    z2 = jnp.zeros((_WCH, _FH), jnp.float32)
    z1 = jnp.zeros((_WCH, _DW), jnp.float32)
    ones1 = jnp.ones((_CHD, _DW), jnp.float32)
    tr = jnp.asarray(training, jnp.float32).reshape(1, 1)
    ba2 = ba.reshape(1, 1)

    u = jax.random.uniform(jax.random.key(42), (_N, 1),
                           minval=1e-10, maxval=1.0 - 1e-10)
    noise = jnp.log(u) - jnp.log(1.0 - u)

    (degp,) = _sc_deg(dst3d, z1, ones1)
    dis, xs = _tc_scale(degp[0, :, :1], degp[1, :, :1], inputs)
    disf = dis.reshape(_N)

    a1, norm3 = _sc_agg(src3, dst3, xs, z2, disf)
    h1s = _tc_layer(a1, dis, W1)
    a2, _ = _sc_agg(src3, dst3, h1s, z2, disf)
    att, q, xq, info = _tc_att(a2, dis, W2, Wa, ba2, noise, tr, inputs)
    attf = att.reshape(_N)
    a3, eatt3 = _sc_agg(src3, dst3, xq, z2, attf)
    s1q = _tc_layer(a3, q, W3)
    a4, _ = _sc_agg(src3, dst3, s1q, z2, disf)
    sp_emb = _tc_pool(a4, q, W4, batch.reshape(_N, 1))

    edge_att = eatt3.reshape(_NS, _EPTP)[:, :_EPT].reshape(_E, 1)
    edge_weights = norm3.reshape(_NS, _EPTP)[:, :_EPT].reshape(_E)
    info_loss = info[0, 0]
    feat_weights = jnp.ones((_F,), jnp.float32)
    return edge_att, info_loss, sp_emb, edge_weights, feat_weights


# speedup vs baseline: 1.2278x; 1.2278x over previous
import functools
import jax
import jax.numpy as jnp
from jax import lax
from jax.experimental import pallas as pl
from jax.experimental.pallas import tpu as pltpu
from jax.experimental.pallas import tpu_sc as plsc

_N = 10000
_E = 320000
_F = 128
_FH = _F // 2
_G = 128

_NC = 2
_NS = 16
_NW = _NC * _NS
_CH = 80
_EPT = _E // _NS
_NCH = 250
_EPTP = _NCH * _CH
_TRASH = _N
_NTR = 512
_NA = _N + _NTR
_EG = 28
_NEG = _NCH // _EG
_CHD = 80
_EPW = _E // _NW
_NCHD = _EPW // _CHD
_TPR = 624
_WCH = 104
_NWC = _TPR // _WCH
_TAIL = _N - _NS * _TPR
_BLK = 1000
_NBLK = _N // _BLK


def _mesh():
    return plsc.VectorSubcoreMesh(core_axis_name="c", subcore_axis_name="s")


_SC_PARAMS = pltpu.CompilerParams(needs_layout_passes=False,
                                  use_tc_tiling_on_sc=False)


_DW = 16


@functools.partial(
    pl.kernel,
    mesh=_mesh(),
    compiler_params=_SC_PARAMS,
    out_type=[jax.ShapeDtypeStruct((_NC, _N, _DW), jnp.float32)],
    scratch_types=[
        pltpu.VMEM((_NCHD, _CHD), jnp.int32),
        pltpu.VMEM((_CHD, _DW), jnp.float32),
        pltpu.VMEM((_WCH, _DW), jnp.float32),
        pltpu.SemaphoreType.DMA,
        pltpu.VMEM_SHARED((_N, _DW), jnp.float32),
    ],
)
def _sc_deg(dst3_hbm, z1_hbm, ones_hbm, deg_out, dst2_v, ones_v, zst_v, sem,
            acc):
    c = lax.axis_index("c")
    s = lax.axis_index("s")
    w = c * _NS + s
    pltpu.sync_copy(z1_hbm, zst_v)
    pltpu.sync_copy(ones_hbm, ones_v)
    pltpu.sync_copy(dst3_hbm.at[w], dst2_v)
    for j in range(_NWC):
        pltpu.sync_copy(zst_v, acc.at[pl.ds(s * _TPR + j * _WCH, _WCH)])

    @pl.when(s == 0)
    def _():
        pltpu.sync_copy(zst_v.at[pl.ds(0, _TAIL)],
                        acc.at[pl.ds(_NS * _TPR, _TAIL)])

    plsc.subcore_barrier()

    def fire(i, carry):
        pltpu.async_copy(ones_v, acc.at[dst2_v.at[i]], sem, add=True)
        return carry

    lax.fori_loop(0, _NCHD, fire, 0)

    def drain(i, carry):
        pltpu.make_async_copy(ones_v, acc.at[dst2_v.at[i]], sem).wait()
        return carry

    lax.fori_loop(0, _NCHD, drain, 0)
    plsc.subcore_barrier()
    for j in range(_NWC):
        r0 = s * _TPR + j * _WCH
        pltpu.sync_copy(acc.at[pl.ds(r0, _WCH)], zst_v)
        pltpu.sync_copy(zst_v, deg_out.at[c, pl.ds(r0, _WCH)])

    @pl.when(s == 0)
    def _():
        pltpu.sync_copy(acc.at[pl.ds(_NS * _TPR, _TAIL)],
                        zst_v.at[pl.ds(0, _TAIL)])
        pltpu.sync_copy(zst_v.at[pl.ds(0, _TAIL)],
                        deg_out.at[c, pl.ds(_NS * _TPR, _TAIL)])


@functools.partial(
    pl.kernel,
    mesh=_mesh(),
    compiler_params=_SC_PARAMS,
    out_type=[jax.ShapeDtypeStruct((_NC, _N, _FH), jnp.float32),
              jax.ShapeDtypeStruct((_NS, _NCH, _CH), jnp.float32)],
    scratch_types=[
        pltpu.VMEM((_NCH, _CH), jnp.int32),
        pltpu.VMEM((_NCH, _CH), jnp.int32),
        pltpu.VMEM((_CH, _FH), jnp.float32),
        pltpu.VMEM((_CH, _FH), jnp.float32),
        pltpu.VMEM((_WCH, _FH), jnp.float32),
        pltpu.SemaphoreType.DMA,
        pltpu.SemaphoreType.DMA,
        pltpu.VMEM((_NA,), jnp.float32),
        pltpu.VMEM((_NCH, _CH), jnp.float32),
        pltpu.VMEM_SHARED((_NA, _FH), jnp.float32),
    ],
)
def _sc_agg(src3, dst3, x_hbm, z_hbm, svec_hbm, out_hbm, eout_hbm,
            src2, dst2, rb0, rb1, stage, sg0, sg1, svec_v, ebuf, acc):
    c = lax.axis_index("c")
    s = lax.axis_index("s")
    rows = [rb0, rb1]
    gsem = [sg0, sg1]

    pltpu.sync_copy(src3.at[s], src2)
    pltpu.sync_copy(dst3.at[s], dst2)
    xh = x_hbm.at[c]
    pltpu.async_copy(xh.at[src2.at[0]], rows[0], gsem[0])
    pltpu.async_copy(xh.at[src2.at[1]], rows[1], gsem[1])

    pltpu.sync_copy(z_hbm, stage)
    for j in range(_NWC):
        pltpu.sync_copy(stage, acc.at[pl.ds(s * _TPR + j * _WCH, _WCH)])

    @pl.when(s == 0)
    def _():
        pltpu.sync_copy(stage.at[pl.ds(0, _TAIL)],
                        acc.at[pl.ds(_NS * _TPR, _TAIL)])

    @pl.when(c == 0)
    def _():
        pltpu.sync_copy(svec_hbm, svec_v.at[pl.ds(0, _N)])

    plsc.subcore_barrier()

    def edge_scalars(ci, er):
        for j2 in range(_CH // 16):
            si = src2[ci, pl.ds(j2 * 16, 16)]
            di = dst2[ci, pl.ds(j2 * 16, 16)]
            gs = plsc.load_gather(svec_v, [si])
            gd = plsc.load_gather(svec_v, [di])
            ebuf[er, pl.ds(j2 * 16, 16)] = gs * gd

    def pair(k, carry):
        for j in range(2):
            ci = 2 * k + j

            @pl.when(c == 0)
            def _():
                edge_scalars(ci, ci)

            pltpu.make_async_copy(xh.at[src2.at[ci]], rows[j],
                                  gsem[j]).wait()
            pltpu.sync_copy(rows[j], acc.at[dst2.at[ci]], add=True)

            @pl.when(ci + 2 < _NCH)
            def _():
                pltpu.async_copy(xh.at[src2.at[ci + 2]], rows[j], gsem[j])
        return carry

    lax.fori_loop(0, _NCH // 2, pair, 0)

    @pl.when(c == 0)
    def _():
        pltpu.sync_copy(ebuf, eout_hbm.at[s])

    plsc.subcore_barrier()
    for j in range(_NWC):
        r0 = s * _TPR + j * _WCH
        pltpu.sync_copy(acc.at[pl.ds(r0, _WCH)], stage)
        pltpu.sync_copy(stage, out_hbm.at[c, pl.ds(r0, _WCH)])

    @pl.when(s == 0)
    def _():
        pltpu.sync_copy(acc.at[pl.ds(_NS * _TPR, _TAIL)],
                        stage.at[pl.ds(0, _TAIL)])
        pltpu.sync_copy(stage.at[pl.ds(0, _TAIL)],
                        out_hbm.at[c, pl.ds(_NS * _TPR, _TAIL)])


def _row_spec(width):
    return pl.BlockSpec((_BLK, width), lambda i: (i, 0))


def _split_spec():
    return pl.BlockSpec((_NC, _BLK, _FH), lambda i: (0, i, 0))


def _full_spec(shape):
    nd = len(shape)
    return pl.BlockSpec(shape, lambda i: (0,) * nd)


def _split_store(ref, y):
    ref[0] = y[:, :_FH]
    ref[1] = y[:, _FH:]


def _split_load(ref):
    return jnp.concatenate([ref[0], ref[1]], axis=-1)


def _tc_scale_body(da_ref, db_ref, x_ref, dis_ref, xs_ref):
    deg = da_ref[...] + db_ref[...]
    dis = lax.rsqrt(jnp.maximum(deg, 1.0))
    dis_ref[...] = dis
    _split_store(xs_ref, dis * x_ref[...])


def _tc_scale(dega, degb, x):
    return pl.pallas_call(
        _tc_scale_body,
        grid=(_NBLK,),
        in_specs=[_row_spec(1), _row_spec(1), _row_spec(_F)],
        out_specs=[_row_spec(1), _split_spec()],
        out_shape=[jax.ShapeDtypeStruct((_N, 1), jnp.float32),
                   jax.ShapeDtypeStruct((_NC, _N, _FH), jnp.float32)],
    )(dega, degb, x)


def _tc_layer_body(p_ref, s_ref, w_ref, y_ref):
    p = _split_load(p_ref)
    sv = s_ref[...]
    h = jnp.maximum(jnp.dot(sv * p, w_ref[...],
                            preferred_element_type=jnp.float32), 0.0)
    _split_store(y_ref, sv * h)


def _tc_layer(part, sv, w):
    return pl.pallas_call(
        _tc_layer_body,
        grid=(_NBLK,),
        in_specs=[_split_spec(), _row_spec(1), _full_spec((_F, _F))],
        out_specs=_split_spec(),
        out_shape=jax.ShapeDtypeStruct((_NC, _N, _FH), jnp.float32),
    )(part, sv, w)


def _tc_att_body(p_ref, dis_ref, w2_ref, wa_ref, ba_ref, nz_ref, tr_ref,
                 x_ref, att_ref, q_ref, xq_ref, info_ref, acc_ref):
    i = pl.program_id(0)
    p = _split_load(p_ref)
    dis = dis_ref[...]
    emb = jnp.maximum(jnp.dot(dis * p, w2_ref[...],
                              preferred_element_type=jnp.float32), 0.0)
    logits = jnp.dot(emb, wa_ref[...],
                     preferred_element_type=jnp.float32) + ba_ref[...]
    att = jax.nn.sigmoid(logits + jnp.where(tr_ref[...] != 0.0,
                                            nz_ref[...], 0.0))
    att_ref[...] = att
    q = dis * att
    q_ref[...] = q
    _split_store(xq_ref, q * x_ref[...])
    r = 0.7
    f = (att * jnp.log(att / r + 1e-6)
         + (1.0 - att) * jnp.log((1.0 - att) / (1.0 - r + 1e-6) + 1e-6))
    part = jnp.sum(f).reshape(1, 1)
    acc_ref[...] = jnp.where(i == 0, part, acc_ref[...] + part)

    @pl.when(i == _NBLK - 1)
    def _():
        info_ref[...] = acc_ref[...] / float(_N)


def _tc_att(part, dis, w2, wa, ba, noise, tr, x):
    return pl.pallas_call(
        _tc_att_body,
        grid=(_NBLK,),
        in_specs=[_split_spec(), _row_spec(1), _full_spec((_F, _F)),
                  _full_spec((_F, 1)), _full_spec((1, 1)), _row_spec(1),
                  _full_spec((1, 1)), _row_spec(_F)],
        out_specs=[_row_spec(1), _row_spec(1), _split_spec(),
                   _full_spec((1, 1))],
        out_shape=[jax.ShapeDtypeStruct((_N, 1), jnp.float32),
                   jax.ShapeDtypeStruct((_N, 1), jnp.float32),
                   jax.ShapeDtypeStruct((_NC, _N, _FH), jnp.float32),
                   jax.ShapeDtypeStruct((1, 1), jnp.float32)],
        scratch_shapes=[pltpu.VMEM((1, 1), jnp.float32)],
    )(part, dis, w2, wa, ba, noise, tr, x)


def _tc_pool_body(p_ref, q_ref, w4_ref, b_ref, sp_ref, pool_ref, cnt_ref):
    i = pl.program_id(0)
    p = _split_load(p_ref)
    s2 = jnp.maximum(q_ref[...] * jnp.dot(p, w4_ref[...],
                                          preferred_element_type=jnp.float32),
                     0.0)
    b = b_ref[...]
    iota = lax.broadcasted_iota(jnp.int32, (_BLK, _G), 1)
    m = (b == iota).astype(jnp.float32)
    dims = (((0,), (0,)), ((), ()))
    pool_d = lax.dot_general(m, s2, dims, preferred_element_type=jnp.float32)
    cnt_d = lax.dot_general(m, jnp.ones((_BLK, 1), jnp.float32), dims,
                            preferred_element_type=jnp.float32)

    @pl.when(i == 0)
    def _():
        pool_ref[...] = pool_d
        cnt_ref[...] = cnt_d

    @pl.when(i > 0)
    def _():
        pool_ref[...] += pool_d
        cnt_ref[...] += cnt_d

    @pl.when(i == _NBLK - 1)
    def _():
        sp_ref[...] = pool_ref[...] / jnp.maximum(cnt_ref[...], 1.0)


def _tc_pool(part, q, w4, batch2d):
    return pl.pallas_call(
        _tc_pool_body,
        grid=(_NBLK,),
        in_specs=[_split_spec(), _row_spec(1), _full_spec((_F, _F)),
                  _row_spec(1)],
        out_specs=_full_spec((_G, _F)),
        out_shape=jax.ShapeDtypeStruct((_G, _F), jnp.float32),
        scratch_shapes=[pltpu.VMEM((_G, _F), jnp.float32),
                        pltpu.VMEM((_G, 1), jnp.float32)],
    )(part, q, w4, batch2d)


def kernel(edge_index, inputs, epoch, training, batch, W1, W2, Wa, ba, W3, W4):
    pad = _EPTP - _EPT
    if pad:
        src3 = jnp.pad(edge_index[0].reshape(_NS, _EPT),
                       ((0, 0), (0, pad))).reshape(_NS, _NCH, _CH)
        padv = _TRASH + (jnp.arange(pad, dtype=jnp.int32) % _NTR)
        dst3 = jnp.concatenate(
            [edge_index[1].reshape(_NS, _EPT),
             jnp.broadcast_to(padv, (_NS, pad))],
            axis=1).reshape(_NS, _NCH, _CH)
    else:
        src3 = edge_index[0].reshape(_NS, _NCH, _CH)
        dst3 = edge_index[1].reshape(_NS, _NCH, _CH)
    dst3d = edge_index[1].reshape(_NW, _NCHD, _CHD)
    z2 = jnp.zeros((_WCH, _FH), jnp.float32)
    z1 = jnp.zeros((_WCH, _DW), jnp.float32)
    ones1 = jnp.ones((_CHD, _DW), jnp.float32)
    tr = jnp.asarray(training, jnp.float32).reshape(1, 1)
    ba2 = ba.reshape(1, 1)

    u = jax.random.uniform(jax.random.key(42), (_N, 1),
                           minval=1e-10, maxval=1.0 - 1e-10)
    noise = jnp.log(u) - jnp.log(1.0 - u)

    (degp,) = _sc_deg(dst3d, z1, ones1)
    dis, xs = _tc_scale(degp[0, :, :1], degp[1, :, :1], inputs)
    disf = dis.reshape(_N)

    a1, norm3 = _sc_agg(src3, dst3, xs, z2, disf)
    h1s = _tc_layer(a1, dis, W1)
    a2, _ = _sc_agg(src3, dst3, h1s, z2, disf)
    att, q, xq, info = _tc_att(a2, dis, W2, Wa, ba2, noise, tr, inputs)
    attf = att.reshape(_N)
    a3, eatt3 = _sc_agg(src3, dst3, xq, z2, attf)
    s1q = _tc_layer(a3, q, W3)
    a4, _ = _sc_agg(src3, dst3, s1q, z2, disf)
    sp_emb = _tc_pool(a4, q, W4, batch.reshape(_N, 1))

    edge_att = eatt3.reshape(_NS, _EPTP)[:, :_EPT].reshape(_E, 1)
    edge_weights = norm3.reshape(_NS, _EPTP)[:, :_EPT].reshape(_E)
    info_loss = info[0, 0]
    feat_weights = jnp.ones((_F,), jnp.float32)
    return edge_att, info_loss, sp_emb, edge_weights, feat_weights


# speedup vs baseline: 1.2402x; 1.0101x over previous
import functools
import jax
import jax.numpy as jnp
from jax import lax
from jax.experimental import pallas as pl
from jax.experimental.pallas import tpu as pltpu
from jax.experimental.pallas import tpu_sc as plsc

_N = 10000
_E = 320000
_F = 128
_FH = _F // 2
_G = 128

_NC = 2
_NS = 16
_NW = _NC * _NS
_CH = 80
_EPT = _E // _NS
_NCH = 250
_EPTP = _NCH * _CH
_TRASH = _N
_NTR = 512
_NA = _N + _NTR
_EG = 28
_NEG = _NCH // _EG
_CHD = 80
_EPW = _E // _NW
_NCHD = _EPW // _CHD
_TPR = 624
_WCH = 104
_NWC = _TPR // _WCH
_TAIL = _N - _NS * _TPR
_BLK = 1000
_NBLK = _N // _BLK


def _mesh():
    return plsc.VectorSubcoreMesh(core_axis_name="c", subcore_axis_name="s")


_SC_PARAMS = pltpu.CompilerParams(needs_layout_passes=False,
                                  use_tc_tiling_on_sc=False)


_DW = 16


@functools.partial(
    pl.kernel,
    mesh=_mesh(),
    compiler_params=_SC_PARAMS,
    out_type=[jax.ShapeDtypeStruct((_NC, _N, _DW), jnp.float32)],
    scratch_types=[
        pltpu.VMEM((_NCHD, _CHD), jnp.int32),
        pltpu.VMEM((_CHD, _DW), jnp.float32),
        pltpu.VMEM((_WCH, _DW), jnp.float32),
        pltpu.SemaphoreType.DMA,
        pltpu.VMEM_SHARED((_N, _DW), jnp.float32),
    ],
)
def _sc_deg(dst3_hbm, z1_hbm, ones_hbm, deg_out, dst2_v, ones_v, zst_v, sem,
            acc):
    c = lax.axis_index("c")
    s = lax.axis_index("s")
    w = c * _NS + s
    pltpu.sync_copy(z1_hbm, zst_v)
    pltpu.sync_copy(ones_hbm, ones_v)
    pltpu.sync_copy(dst3_hbm.at[w], dst2_v)
    for j in range(_NWC):
        pltpu.sync_copy(zst_v, acc.at[pl.ds(s * _TPR + j * _WCH, _WCH)])

    @pl.when(s == 0)
    def _():
        pltpu.sync_copy(zst_v.at[pl.ds(0, _TAIL)],
                        acc.at[pl.ds(_NS * _TPR, _TAIL)])

    plsc.subcore_barrier()

    def fire(i, carry):
        pltpu.async_copy(ones_v, acc.at[dst2_v.at[i]], sem, add=True)
        return carry

    lax.fori_loop(0, _NCHD, fire, 0)

    def drain(i, carry):
        pltpu.make_async_copy(ones_v, acc.at[dst2_v.at[i]], sem).wait()
        return carry

    lax.fori_loop(0, _NCHD, drain, 0)
    plsc.subcore_barrier()
    for j in range(_NWC):
        r0 = s * _TPR + j * _WCH
        pltpu.sync_copy(acc.at[pl.ds(r0, _WCH)], zst_v)
        pltpu.sync_copy(zst_v, deg_out.at[c, pl.ds(r0, _WCH)])

    @pl.when(s == 0)
    def _():
        pltpu.sync_copy(acc.at[pl.ds(_NS * _TPR, _TAIL)],
                        zst_v.at[pl.ds(0, _TAIL)])
        pltpu.sync_copy(zst_v.at[pl.ds(0, _TAIL)],
                        deg_out.at[c, pl.ds(_NS * _TPR, _TAIL)])


@functools.partial(
    pl.kernel,
    mesh=_mesh(),
    compiler_params=_SC_PARAMS,
    out_type=[jax.ShapeDtypeStruct((_NC, _N, _FH), jnp.float32),
              jax.ShapeDtypeStruct((_NS, _NCH, _CH), jnp.float32)],
    scratch_types=[
        pltpu.VMEM((_NCH, _CH), jnp.int32),
        pltpu.VMEM((_NCH, _CH), jnp.int32),
        pltpu.VMEM((_CH, _FH), jnp.float32),
        pltpu.VMEM((_CH, _FH), jnp.float32),
        pltpu.VMEM((_WCH, _FH), jnp.float32),
        pltpu.SemaphoreType.DMA,
        pltpu.SemaphoreType.DMA,
        pltpu.VMEM((_NA,), jnp.float32),
        pltpu.VMEM((_NCH, _CH), jnp.float32),
        pltpu.VMEM_SHARED((_NA, _FH), jnp.float32),
    ],
)
def _sc_agg(src3, dst3, x_hbm, z_hbm, svec_hbm, out_hbm, eout_hbm,
            src2, dst2, rb0, rb1, stage, sg0, sg1, svec_v, ebuf, acc):
    c = lax.axis_index("c")
    s = lax.axis_index("s")
    rows = [rb0, rb1]
    gsem = [sg0, sg1]

    pltpu.sync_copy(src3.at[s], src2)
    pltpu.sync_copy(dst3.at[s], dst2)
    xh = x_hbm.at[c]
    pltpu.async_copy(xh.at[src2.at[0]], rows[0], gsem[0])
    pltpu.async_copy(xh.at[src2.at[1]], rows[1], gsem[1])

    pltpu.sync_copy(z_hbm, stage)
    for j in range(_NWC):
        pltpu.sync_copy(stage, acc.at[pl.ds(s * _TPR + j * _WCH, _WCH)])

    @pl.when(s == 0)
    def _():
        pltpu.sync_copy(stage.at[pl.ds(0, _TAIL)],
                        acc.at[pl.ds(_NS * _TPR, _TAIL)])

    @pl.when(c == 0)
    def _():
        pltpu.sync_copy(svec_hbm, svec_v.at[pl.ds(0, _N)])

    plsc.subcore_barrier()

    def edge_scalars(ci, er):
        for j2 in range(_CH // 16):
            si = src2[ci, pl.ds(j2 * 16, 16)]
            di = dst2[ci, pl.ds(j2 * 16, 16)]
            gs = plsc.load_gather(svec_v, [si])
            gd = plsc.load_gather(svec_v, [di])
            ebuf[er, pl.ds(j2 * 16, 16)] = gs * gd

    def pair(k, carry):
        for j in range(2):
            ci = 2 * k + j

            @pl.when(c == 0)
            def _():
                edge_scalars(ci, ci)

            pltpu.make_async_copy(xh.at[src2.at[ci]], rows[j],
                                  gsem[j]).wait()
            pltpu.sync_copy(rows[j], acc.at[dst2.at[ci]], add=True)

            @pl.when(ci + 2 < _NCH)
            def _():
                pltpu.async_copy(xh.at[src2.at[ci + 2]], rows[j], gsem[j])
        return carry

    lax.fori_loop(0, _NCH // 2, pair, 0)

    @pl.when(c == 0)
    def _():
        pltpu.sync_copy(ebuf, eout_hbm.at[s])

    plsc.subcore_barrier()
    for j in range(_NWC):
        r0 = s * _TPR + j * _WCH
        pltpu.sync_copy(acc.at[pl.ds(r0, _WCH)], stage)
        pltpu.sync_copy(stage, out_hbm.at[c, pl.ds(r0, _WCH)])

    @pl.when(s == 0)
    def _():
        pltpu.sync_copy(acc.at[pl.ds(_NS * _TPR, _TAIL)],
                        stage.at[pl.ds(0, _TAIL)])
        pltpu.sync_copy(stage.at[pl.ds(0, _TAIL)],
                        out_hbm.at[c, pl.ds(_NS * _TPR, _TAIL)])


def _row_spec(width):
    return pl.BlockSpec((_BLK, width), lambda i: (i, 0))


def _split_spec():
    return pl.BlockSpec((_NC, _BLK, _FH), lambda i: (0, i, 0))


def _full_spec(shape):
    nd = len(shape)
    return pl.BlockSpec(shape, lambda i: (0,) * nd)


def _split_store(ref, y):
    ref[0] = y[:, :_FH]
    ref[1] = y[:, _FH:]


def _split_load(ref):
    return jnp.concatenate([ref[0], ref[1]], axis=-1)


def _tc_scale_body(dp_ref, x_ref, dis_ref, xs_ref):
    deg = dp_ref[0, :, :1] + dp_ref[1, :, :1]
    dis = lax.rsqrt(jnp.maximum(deg, 1.0))
    dis_ref[...] = dis
    _split_store(xs_ref, dis * x_ref[...])


def _tc_scale(degp, x):
    return pl.pallas_call(
        _tc_scale_body,
        grid=(_NBLK,),
        in_specs=[pl.BlockSpec((_NC, _BLK, _DW), lambda i: (0, i, 0)),
                  _row_spec(_F)],
        out_specs=[_row_spec(1), _split_spec()],
        out_shape=[jax.ShapeDtypeStruct((_N, 1), jnp.float32),
                   jax.ShapeDtypeStruct((_NC, _N, _FH), jnp.float32)],
    )(degp, x)


def _tc_layer_body(p_ref, s_ref, w_ref, y_ref):
    p = _split_load(p_ref)
    sv = s_ref[...]
    h = jnp.maximum(jnp.dot(sv * p, w_ref[...],
                            preferred_element_type=jnp.float32), 0.0)
    _split_store(y_ref, sv * h)


def _tc_layer(part, sv, w):
    return pl.pallas_call(
        _tc_layer_body,
        grid=(_NBLK,),
        in_specs=[_split_spec(), _row_spec(1), _full_spec((_F, _F))],
        out_specs=_split_spec(),
        out_shape=jax.ShapeDtypeStruct((_NC, _N, _FH), jnp.float32),
    )(part, sv, w)


def _tc_att_body(p_ref, dis_ref, w2_ref, wa_ref, ba_ref, nz_ref, tr_ref,
                 x_ref, att_ref, q_ref, xq_ref, info_ref, acc_ref):
    i = pl.program_id(0)
    p = _split_load(p_ref)
    dis = dis_ref[...]
    emb = jnp.maximum(jnp.dot(dis * p, w2_ref[...],
                              preferred_element_type=jnp.float32), 0.0)
    logits = jnp.dot(emb, wa_ref[...],
                     preferred_element_type=jnp.float32) + ba_ref[...]
    att = jax.nn.sigmoid(logits + jnp.where(tr_ref[...] != 0.0,
                                            nz_ref[...], 0.0))
    att_ref[...] = att
    q = dis * att
    q_ref[...] = q
    _split_store(xq_ref, q * x_ref[...])
    r = 0.7
    f = (att * jnp.log(att / r + 1e-6)
         + (1.0 - att) * jnp.log((1.0 - att) / (1.0 - r + 1e-6) + 1e-6))
    part = jnp.sum(f).reshape(1, 1)
    acc_ref[...] = jnp.where(i == 0, part, acc_ref[...] + part)

    @pl.when(i == _NBLK - 1)
    def _():
        info_ref[...] = acc_ref[...] / float(_N)


def _tc_att(part, dis, w2, wa, ba, noise, tr, x):
    return pl.pallas_call(
        _tc_att_body,
        grid=(_NBLK,),
        in_specs=[_split_spec(), _row_spec(1), _full_spec((_F, _F)),
                  _full_spec((_F, 1)), _full_spec((1, 1)), _row_spec(1),
                  _full_spec((1, 1)), _row_spec(_F)],
        out_specs=[_row_spec(1), _row_spec(1), _split_spec(),
                   _full_spec((1, 1))],
        out_shape=[jax.ShapeDtypeStruct((_N, 1), jnp.float32),
                   jax.ShapeDtypeStruct((_N, 1), jnp.float32),
                   jax.ShapeDtypeStruct((_NC, _N, _FH), jnp.float32),
                   jax.ShapeDtypeStruct((1, 1), jnp.float32)],
        scratch_shapes=[pltpu.VMEM((1, 1), jnp.float32)],
    )(part, dis, w2, wa, ba, noise, tr, x)


def _tc_pool_body(p_ref, q_ref, w4_ref, b_ref, sp_ref, pool_ref, cnt_ref):
    i = pl.program_id(0)
    p = _split_load(p_ref)
    s2 = jnp.maximum(q_ref[...] * jnp.dot(p, w4_ref[...],
                                          preferred_element_type=jnp.float32),
                     0.0)
    b = b_ref[...]
    iota = lax.broadcasted_iota(jnp.int32, (_BLK, _G), 1)
    m = (b == iota).astype(jnp.float32)
    dims = (((0,), (0,)), ((), ()))
    pool_d = lax.dot_general(m, s2, dims, preferred_element_type=jnp.float32)
    cnt_d = lax.dot_general(m, jnp.ones((_BLK, 1), jnp.float32), dims,
                            preferred_element_type=jnp.float32)

    @pl.when(i == 0)
    def _():
        pool_ref[...] = pool_d
        cnt_ref[...] = cnt_d

    @pl.when(i > 0)
    def _():
        pool_ref[...] += pool_d
        cnt_ref[...] += cnt_d

    @pl.when(i == _NBLK - 1)
    def _():
        sp_ref[...] = pool_ref[...] / jnp.maximum(cnt_ref[...], 1.0)


def _tc_pool(part, q, w4, batch2d):
    return pl.pallas_call(
        _tc_pool_body,
        grid=(_NBLK,),
        in_specs=[_split_spec(), _row_spec(1), _full_spec((_F, _F)),
                  _row_spec(1)],
        out_specs=_full_spec((_G, _F)),
        out_shape=jax.ShapeDtypeStruct((_G, _F), jnp.float32),
        scratch_shapes=[pltpu.VMEM((_G, _F), jnp.float32),
                        pltpu.VMEM((_G, 1), jnp.float32)],
    )(part, q, w4, batch2d)


def kernel(edge_index, inputs, epoch, training, batch, W1, W2, Wa, ba, W3, W4):
    pad = _EPTP - _EPT
    if pad:
        src3 = jnp.pad(edge_index[0].reshape(_NS, _EPT),
                       ((0, 0), (0, pad))).reshape(_NS, _NCH, _CH)
        padv = _TRASH + (jnp.arange(pad, dtype=jnp.int32) % _NTR)
        dst3 = jnp.concatenate(
            [edge_index[1].reshape(_NS, _EPT),
             jnp.broadcast_to(padv, (_NS, pad))],
            axis=1).reshape(_NS, _NCH, _CH)
    else:
        src3 = edge_index[0].reshape(_NS, _NCH, _CH)
        dst3 = edge_index[1].reshape(_NS, _NCH, _CH)
    dst3d = edge_index[1].reshape(_NW, _NCHD, _CHD)
    z2 = jnp.zeros((_WCH, _FH), jnp.float32)
    z1 = jnp.zeros((_WCH, _DW), jnp.float32)
    ones1 = jnp.ones((_CHD, _DW), jnp.float32)
    tr = jnp.asarray(training, jnp.float32).reshape(1, 1)
    ba2 = ba.reshape(1, 1)

    u = jax.random.uniform(jax.random.key(42), (_N, 1),
                           minval=1e-10, maxval=1.0 - 1e-10)
    noise = jnp.log(u) - jnp.log(1.0 - u)

    (degp,) = _sc_deg(dst3d, z1, ones1)
    dis, xs = _tc_scale(degp, inputs)
    disf = dis.reshape(_N)

    a1, norm3 = _sc_agg(src3, dst3, xs, z2, disf)
    h1s = _tc_layer(a1, dis, W1)
    a2, _ = _sc_agg(src3, dst3, h1s, z2, disf)
    att, q, xq, info = _tc_att(a2, dis, W2, Wa, ba2, noise, tr, inputs)
    attf = att.reshape(_N)
    a3, eatt3 = _sc_agg(src3, dst3, xq, z2, attf)
    s1q = _tc_layer(a3, q, W3)
    a4, _ = _sc_agg(src3, dst3, s1q, z2, disf)
    sp_emb = _tc_pool(a4, q, W4, batch.reshape(_N, 1))

    edge_att = eatt3.reshape(_NS, _EPTP)[:, :_EPT].reshape(_E, 1)
    edge_weights = norm3.reshape(_NS, _EPTP)[:, :_EPT].reshape(_E)
    info_loss = info[0, 0]
    feat_weights = jnp.ones((_F,), jnp.float32)
    return edge_att, info_loss, sp_emb, edge_weights, feat_weights


# speedup vs baseline: 1.4114x; 1.1380x over previous
import functools
import jax
import jax.numpy as jnp
from jax import lax
from jax.experimental import pallas as pl
from jax.experimental.pallas import tpu as pltpu
from jax.experimental.pallas import tpu_sc as plsc

_N = 10000
_E = 320000
_F = 128
_FH = _F // 2
_G = 128

_NC = 2
_NS = 16
_NW = _NC * _NS
_CH = 80
_EPT = _E // _NS
_NCH = 250
_EPTP = _NCH * _CH
_TRASH = _N
_NTR = 512
_NA = _N + _NTR
_EG = 28
_NEG = _NCH // _EG
_CHD = 80
_EPW = _E // _NW
_NCHD = _EPW // _CHD
_TPR = 624
_WCH = 104
_NWC = _TPR // _WCH
_TAIL = _N - _NS * _TPR
_BLK = 1000
_NBLK = _N // _BLK


def _mesh():
    return plsc.VectorSubcoreMesh(core_axis_name="c", subcore_axis_name="s")


_SC_PARAMS = pltpu.CompilerParams(needs_layout_passes=False,
                                  use_tc_tiling_on_sc=False)


_DW = 16


@functools.partial(
    pl.kernel,
    mesh=_mesh(),
    compiler_params=_SC_PARAMS,
    out_type=[jax.ShapeDtypeStruct((_NC, _N, _DW), jnp.float32)],
    scratch_types=[
        pltpu.VMEM((_NCHD, _CHD), jnp.int32),
        pltpu.VMEM((_CHD, _DW), jnp.float32),
        pltpu.VMEM((_WCH, _DW), jnp.float32),
        pltpu.SemaphoreType.DMA,
        pltpu.VMEM_SHARED((_N, _DW), jnp.float32),
    ],
)
def _sc_deg(dst3_hbm, z1_hbm, ones_hbm, deg_out, dst2_v, ones_v, zst_v, sem,
            acc):
    c = lax.axis_index("c")
    s = lax.axis_index("s")
    w = c * _NS + s
    pltpu.sync_copy(z1_hbm, zst_v)
    pltpu.sync_copy(ones_hbm, ones_v)
    pltpu.sync_copy(dst3_hbm.at[w], dst2_v)
    for j in range(_NWC):
        pltpu.sync_copy(zst_v, acc.at[pl.ds(s * _TPR + j * _WCH, _WCH)])

    @pl.when(s == 0)
    def _():
        pltpu.sync_copy(zst_v.at[pl.ds(0, _TAIL)],
                        acc.at[pl.ds(_NS * _TPR, _TAIL)])

    plsc.subcore_barrier()

    def fire(i, carry):
        pltpu.async_copy(ones_v, acc.at[dst2_v.at[i]], sem, add=True)
        return carry

    lax.fori_loop(0, _NCHD, fire, 0)

    def drain(i, carry):
        pltpu.make_async_copy(ones_v, acc.at[dst2_v.at[i]], sem).wait()
        return carry

    lax.fori_loop(0, _NCHD, drain, 0)
    plsc.subcore_barrier()
    for j in range(_NWC):
        r0 = s * _TPR + j * _WCH
        pltpu.sync_copy(acc.at[pl.ds(r0, _WCH)], zst_v)
        pltpu.sync_copy(zst_v, deg_out.at[c, pl.ds(r0, _WCH)])

    @pl.when(s == 0)
    def _():
        pltpu.sync_copy(acc.at[pl.ds(_NS * _TPR, _TAIL)],
                        zst_v.at[pl.ds(0, _TAIL)])
        pltpu.sync_copy(zst_v.at[pl.ds(0, _TAIL)],
                        deg_out.at[c, pl.ds(_NS * _TPR, _TAIL)])


@functools.partial(
    pl.kernel,
    mesh=_mesh(),
    compiler_params=_SC_PARAMS,
    out_type=[jax.ShapeDtypeStruct((_NC, _N, _FH), jnp.float32),
              jax.ShapeDtypeStruct((_NS, _NCH, _CH), jnp.float32)],
    scratch_types=[
        pltpu.VMEM((_NCH, _CH), jnp.int32),
        pltpu.VMEM((_NCH, _CH), jnp.int32),
        pltpu.VMEM((_CH, _FH), jnp.float32),
        pltpu.VMEM((_CH, _FH), jnp.float32),
        pltpu.VMEM((_WCH, _FH), jnp.float32),
        pltpu.SemaphoreType.DMA,
        pltpu.SemaphoreType.DMA,
        pltpu.VMEM((_NA,), jnp.float32),
        pltpu.VMEM((_NCH, _CH), jnp.float32),
        pltpu.VMEM_SHARED((_NA, _FH), jnp.float32),
    ],
)
def _sc_agg(src3, dst3, x_hbm, z_hbm, svec_hbm, out_hbm, eout_hbm,
            src2, dst2, rb0, rb1, stage, sg0, sg1, svec_v, ebuf, acc):
    c = lax.axis_index("c")
    s = lax.axis_index("s")
    rows = [rb0, rb1]
    gsem = [sg0, sg1]

    pltpu.sync_copy(src3.at[s], src2)
    pltpu.sync_copy(dst3.at[s], dst2)
    xh = x_hbm.at[c]
    pltpu.async_copy(xh.at[src2.at[0]], rows[0], gsem[0])
    pltpu.async_copy(xh.at[src2.at[1]], rows[1], gsem[1])

    pltpu.sync_copy(z_hbm, stage)
    for j in range(_NWC):
        pltpu.sync_copy(stage, acc.at[pl.ds(s * _TPR + j * _WCH, _WCH)])

    @pl.when(s == 0)
    def _():
        pltpu.sync_copy(stage.at[pl.ds(0, _TAIL)],
                        acc.at[pl.ds(_NS * _TPR, _TAIL)])

    @pl.when(c == 0)
    def _():
        pltpu.sync_copy(svec_hbm, svec_v.at[pl.ds(0, _N)])

    plsc.subcore_barrier()

    def edge_scalars(ci, er):
        for j2 in range(_CH // 16):
            si = src2[ci, pl.ds(j2 * 16, 16)]
            di = dst2[ci, pl.ds(j2 * 16, 16)]
            gs = plsc.load_gather(svec_v, [si])
            gd = plsc.load_gather(svec_v, [di])
            ebuf[er, pl.ds(j2 * 16, 16)] = gs * gd

    def pair(k, carry):
        for j in range(2):
            ci = 2 * k + j

            @pl.when(c == 0)
            def _():
                edge_scalars(ci, ci)

            pltpu.make_async_copy(xh.at[src2.at[ci]], rows[j],
                                  gsem[j]).wait()
            pltpu.sync_copy(rows[j], acc.at[dst2.at[ci]], add=True)

            @pl.when(ci + 2 < _NCH)
            def _():
                pltpu.async_copy(xh.at[src2.at[ci + 2]], rows[j], gsem[j])
        return carry

    lax.fori_loop(0, _NCH // 2, pair, 0)

    @pl.when(c == 0)
    def _():
        pltpu.sync_copy(ebuf, eout_hbm.at[s])

    plsc.subcore_barrier()
    for j in range(_NWC):
        r0 = s * _TPR + j * _WCH
        pltpu.sync_copy(acc.at[pl.ds(r0, _WCH)], stage)
        pltpu.sync_copy(stage, out_hbm.at[c, pl.ds(r0, _WCH)])

    @pl.when(s == 0)
    def _():
        pltpu.sync_copy(acc.at[pl.ds(_NS * _TPR, _TAIL)],
                        stage.at[pl.ds(0, _TAIL)])
        pltpu.sync_copy(stage.at[pl.ds(0, _TAIL)],
                        out_hbm.at[c, pl.ds(_NS * _TPR, _TAIL)])


@functools.partial(
    pl.kernel,
    mesh=_mesh(),
    compiler_params=_SC_PARAMS,
    out_type=[jax.ShapeDtypeStruct((_NC, _N, _FH), jnp.float32)],
    scratch_types=[
        pltpu.VMEM((_NCH, _CH), jnp.int32),
        pltpu.VMEM((_NCH, _CH), jnp.int32),
        pltpu.VMEM((_CH, _FH), jnp.float32),
        pltpu.VMEM((_CH, _FH), jnp.float32),
        pltpu.VMEM((_CH, _FH), jnp.float32),
        pltpu.VMEM((_CH, _FH), jnp.float32),
        pltpu.VMEM((_WCH, _FH), jnp.float32),
        pltpu.SemaphoreType.DMA,
        pltpu.SemaphoreType.DMA,
        pltpu.SemaphoreType.DMA,
        pltpu.SemaphoreType.DMA,
        pltpu.SemaphoreType.DMA,
        pltpu.SemaphoreType.DMA,
        pltpu.SemaphoreType.DMA,
        pltpu.SemaphoreType.DMA,
        pltpu.VMEM_SHARED((_NA, _FH), jnp.float32),
    ],
)
def _sc_agg_plain(src3, dst3, x_hbm, z_hbm, out_hbm,
                  src2, dst2, rb0, rb1, rb2, rb3, stage,
                  sg0, sg1, sg2, sg3, ss0, ss1, ss2, ss3, acc):
    c = lax.axis_index("c")
    s = lax.axis_index("s")
    rows = [rb0, rb1, rb2, rb3]
    gsem = [sg0, sg1, sg2, sg3]
    ssem = [ss0, ss1, ss2, ss3]

    pltpu.sync_copy(src3.at[s], src2)
    pltpu.sync_copy(dst3.at[s], dst2)
    xh = x_hbm.at[c]
    pltpu.async_copy(xh.at[src2.at[0]], rows[0], gsem[0])
    pltpu.async_copy(xh.at[src2.at[1]], rows[1], gsem[1])

    pltpu.sync_copy(z_hbm, stage)
    for j in range(_NWC):
        pltpu.sync_copy(stage, acc.at[pl.ds(s * _TPR + j * _WCH, _WCH)])

    @pl.when(s == 0)
    def _():
        pltpu.sync_copy(stage.at[pl.ds(0, _TAIL)],
                        acc.at[pl.ds(_NS * _TPR, _TAIL)])

    plsc.subcore_barrier()

    def chunk_step(ci, j):
        bp = (j + 2) % 4

        @pl.when(ci >= 2)
        def _():
            pltpu.make_async_copy(rows[bp], acc.at[dst2.at[ci - 2]],
                                  ssem[bp]).wait()

        @pl.when(ci + 2 < _NCH)
        def _():
            pltpu.async_copy(xh.at[src2.at[ci + 2]], rows[bp], gsem[bp])

        pltpu.make_async_copy(xh.at[src2.at[ci]], rows[j], gsem[j]).wait()
        pltpu.async_copy(rows[j], acc.at[dst2.at[ci]], ssem[j], add=True)

    def quad(k, carry):
        for j in range(4):
            chunk_step(4 * k + j, j)
        return carry

    lax.fori_loop(0, _NCH // 4, quad, 0)
    chunk_step(_NCH - 2, 0)
    chunk_step(_NCH - 1, 1)
    pltpu.make_async_copy(rows[0], acc.at[dst2.at[_NCH - 2]], ssem[0]).wait()
    pltpu.make_async_copy(rows[1], acc.at[dst2.at[_NCH - 1]], ssem[1]).wait()

    plsc.subcore_barrier()
    for j in range(_NWC):
        r0 = s * _TPR + j * _WCH
        pltpu.sync_copy(acc.at[pl.ds(r0, _WCH)], stage)
        pltpu.sync_copy(stage, out_hbm.at[c, pl.ds(r0, _WCH)])

    @pl.when(s == 0)
    def _():
        pltpu.sync_copy(acc.at[pl.ds(_NS * _TPR, _TAIL)],
                        stage.at[pl.ds(0, _TAIL)])
        pltpu.sync_copy(stage.at[pl.ds(0, _TAIL)],
                        out_hbm.at[c, pl.ds(_NS * _TPR, _TAIL)])


def _row_spec(width):
    return pl.BlockSpec((_BLK, width), lambda i: (i, 0))


def _split_spec():
    return pl.BlockSpec((_NC, _BLK, _FH), lambda i: (0, i, 0))


def _full_spec(shape):
    nd = len(shape)
    return pl.BlockSpec(shape, lambda i: (0,) * nd)


def _split_store(ref, y):
    ref[0] = y[:, :_FH]
    ref[1] = y[:, _FH:]


def _split_load(ref):
    return jnp.concatenate([ref[0], ref[1]], axis=-1)


def _tc_scale_body(dp_ref, x_ref, dis_ref, xs_ref):
    deg = dp_ref[0, :, :1] + dp_ref[1, :, :1]
    dis = lax.rsqrt(jnp.maximum(deg, 1.0))
    dis_ref[...] = dis
    _split_store(xs_ref, dis * x_ref[...])


def _tc_scale(degp, x):
    return pl.pallas_call(
        _tc_scale_body,
        grid=(_NBLK,),
        in_specs=[pl.BlockSpec((_NC, _BLK, _DW), lambda i: (0, i, 0)),
                  _row_spec(_F)],
        out_specs=[_row_spec(1), _split_spec()],
        out_shape=[jax.ShapeDtypeStruct((_N, 1), jnp.float32),
                   jax.ShapeDtypeStruct((_NC, _N, _FH), jnp.float32)],
    )(degp, x)


def _tc_layer_body(p_ref, s_ref, w_ref, y_ref):
    p = _split_load(p_ref)
    sv = s_ref[...]
    h = jnp.maximum(jnp.dot(sv * p, w_ref[...],
                            preferred_element_type=jnp.float32), 0.0)
    _split_store(y_ref, sv * h)


def _tc_layer(part, sv, w):
    return pl.pallas_call(
        _tc_layer_body,
        grid=(_NBLK,),
        in_specs=[_split_spec(), _row_spec(1), _full_spec((_F, _F))],
        out_specs=_split_spec(),
        out_shape=jax.ShapeDtypeStruct((_NC, _N, _FH), jnp.float32),
    )(part, sv, w)


def _tc_att_body(p_ref, dis_ref, w2_ref, wa_ref, ba_ref, nz_ref, tr_ref,
                 x_ref, att_ref, q_ref, xq_ref, info_ref, acc_ref):
    i = pl.program_id(0)
    p = _split_load(p_ref)
    dis = dis_ref[...]
    emb = jnp.maximum(jnp.dot(dis * p, w2_ref[...],
                              preferred_element_type=jnp.float32), 0.0)
    logits = jnp.dot(emb, wa_ref[...],
                     preferred_element_type=jnp.float32) + ba_ref[...]
    att = jax.nn.sigmoid(logits + jnp.where(tr_ref[...] != 0.0,
                                            nz_ref[...], 0.0))
    att_ref[...] = att
    q = dis * att
    q_ref[...] = q
    _split_store(xq_ref, q * x_ref[...])
    r = 0.7
    f = (att * jnp.log(att / r + 1e-6)
         + (1.0 - att) * jnp.log((1.0 - att) / (1.0 - r + 1e-6) + 1e-6))
    part = jnp.sum(f).reshape(1, 1)
    acc_ref[...] = jnp.where(i == 0, part, acc_ref[...] + part)

    @pl.when(i == _NBLK - 1)
    def _():
        info_ref[...] = acc_ref[...] / float(_N)


def _tc_att(part, dis, w2, wa, ba, noise, tr, x):
    return pl.pallas_call(
        _tc_att_body,
        grid=(_NBLK,),
        in_specs=[_split_spec(), _row_spec(1), _full_spec((_F, _F)),
                  _full_spec((_F, 1)), _full_spec((1, 1)), _row_spec(1),
                  _full_spec((1, 1)), _row_spec(_F)],
        out_specs=[_row_spec(1), _row_spec(1), _split_spec(),
                   _full_spec((1, 1))],
        out_shape=[jax.ShapeDtypeStruct((_N, 1), jnp.float32),
                   jax.ShapeDtypeStruct((_N, 1), jnp.float32),
                   jax.ShapeDtypeStruct((_NC, _N, _FH), jnp.float32),
                   jax.ShapeDtypeStruct((1, 1), jnp.float32)],
        scratch_shapes=[pltpu.VMEM((1, 1), jnp.float32)],
    )(part, dis, w2, wa, ba, noise, tr, x)


def _tc_pool_body(p_ref, q_ref, w4_ref, b_ref, sp_ref, pool_ref, cnt_ref):
    i = pl.program_id(0)
    p = _split_load(p_ref)
    s2 = jnp.maximum(q_ref[...] * jnp.dot(p, w4_ref[...],
                                          preferred_element_type=jnp.float32),
                     0.0)
    b = b_ref[...]
    iota = lax.broadcasted_iota(jnp.int32, (_BLK, _G), 1)
    m = (b == iota).astype(jnp.float32)
    dims = (((0,), (0,)), ((), ()))
    pool_d = lax.dot_general(m, s2, dims, preferred_element_type=jnp.float32)
    cnt_d = lax.dot_general(m, jnp.ones((_BLK, 1), jnp.float32), dims,
                            preferred_element_type=jnp.float32)

    @pl.when(i == 0)
    def _():
        pool_ref[...] = pool_d
        cnt_ref[...] = cnt_d

    @pl.when(i > 0)
    def _():
        pool_ref[...] += pool_d
        cnt_ref[...] += cnt_d

    @pl.when(i == _NBLK - 1)
    def _():
        sp_ref[...] = pool_ref[...] / jnp.maximum(cnt_ref[...], 1.0)


def _tc_pool(part, q, w4, batch2d):
    return pl.pallas_call(
        _tc_pool_body,
        grid=(_NBLK,),
        in_specs=[_split_spec(), _row_spec(1), _full_spec((_F, _F)),
                  _row_spec(1)],
        out_specs=_full_spec((_G, _F)),
        out_shape=jax.ShapeDtypeStruct((_G, _F), jnp.float32),
        scratch_shapes=[pltpu.VMEM((_G, _F), jnp.float32),
                        pltpu.VMEM((_G, 1), jnp.float32)],
    )(part, q, w4, batch2d)


def kernel(edge_index, inputs, epoch, training, batch, W1, W2, Wa, ba, W3, W4):
    pad = _EPTP - _EPT
    if pad:
        src3 = jnp.pad(edge_index[0].reshape(_NS, _EPT),
                       ((0, 0), (0, pad))).reshape(_NS, _NCH, _CH)
        padv = _TRASH + (jnp.arange(pad, dtype=jnp.int32) % _NTR)
        dst3 = jnp.concatenate(
            [edge_index[1].reshape(_NS, _EPT),
             jnp.broadcast_to(padv, (_NS, pad))],
            axis=1).reshape(_NS, _NCH, _CH)
    else:
        src3 = edge_index[0].reshape(_NS, _NCH, _CH)
        dst3 = edge_index[1].reshape(_NS, _NCH, _CH)
    dst3d = edge_index[1].reshape(_NW, _NCHD, _CHD)
    z2 = jnp.zeros((_WCH, _FH), jnp.float32)
    z1 = jnp.zeros((_WCH, _DW), jnp.float32)
    ones1 = jnp.ones((_CHD, _DW), jnp.float32)
    tr = jnp.asarray(training, jnp.float32).reshape(1, 1)
    ba2 = ba.reshape(1, 1)

    u = jax.random.uniform(jax.random.key(42), (_N, 1),
                           minval=1e-10, maxval=1.0 - 1e-10)
    noise = jnp.log(u) - jnp.log(1.0 - u)

    (degp,) = _sc_deg(dst3d, z1, ones1)
    dis, xs = _tc_scale(degp, inputs)
    disf = dis.reshape(_N)

    a1, norm3 = _sc_agg(src3, dst3, xs, z2, disf)
    h1s = _tc_layer(a1, dis, W1)
    (a2,) = _sc_agg_plain(src3, dst3, h1s, z2)
    att, q, xq, info = _tc_att(a2, dis, W2, Wa, ba2, noise, tr, inputs)
    attf = att.reshape(_N)
    a3, eatt3 = _sc_agg(src3, dst3, xq, z2, attf)
    s1q = _tc_layer(a3, q, W3)
    (a4,) = _sc_agg_plain(src3, dst3, s1q, z2)
    sp_emb = _tc_pool(a4, q, W4, batch.reshape(_N, 1))

    edge_att = eatt3.reshape(_NS, _EPTP)[:, :_EPT].reshape(_E, 1)
    edge_weights = norm3.reshape(_NS, _EPTP)[:, :_EPT].reshape(_E)
    info_loss = info[0, 0]
    feat_weights = jnp.ones((_F,), jnp.float32)
    return edge_att, info_loss, sp_emb, edge_weights, feat_weights


# speedup vs baseline: 1.5062x; 1.0672x over previous
import functools
import jax
import jax.numpy as jnp
from jax import lax
from jax.experimental import pallas as pl
from jax.experimental.pallas import tpu as pltpu
from jax.experimental.pallas import tpu_sc as plsc

_N = 10000
_E = 320000
_F = 128
_FH = _F // 2
_G = 128

_NC = 2
_NS = 16
_NW = _NC * _NS
_CH = 80
_EPT = _E // _NS
_NCH = 250
_EPTP = _NCH * _CH
_TRASH = _N
_NTR = 16
_NA = _N
_EG = 28
_NEG = _NCH // _EG
_CHD = 80
_EPW = _E // _NW
_NCHD = _EPW // _CHD
_TPR = 624
_WCH = 104
_NWC = _TPR // _WCH
_WCHF = 48
_NWCF = _TPR // _WCHF
_TAIL = _N - _NS * _TPR
_BLK = 1000
_NBLK = _N // _BLK


def _mesh():
    return plsc.VectorSubcoreMesh(core_axis_name="c", subcore_axis_name="s")


_SC_PARAMS = pltpu.CompilerParams(needs_layout_passes=False,
                                  use_tc_tiling_on_sc=False)


_DW = 16


@functools.partial(
    pl.kernel,
    mesh=_mesh(),
    compiler_params=_SC_PARAMS,
    out_type=[jax.ShapeDtypeStruct((_NC, _N, _DW), jnp.float32)],
    scratch_types=[
        pltpu.VMEM((_NCHD, _CHD), jnp.int32),
        pltpu.VMEM((_CHD, _DW), jnp.float32),
        pltpu.VMEM((_WCH, _DW), jnp.float32),
        pltpu.SemaphoreType.DMA,
        pltpu.VMEM_SHARED((_N, _DW), jnp.float32),
    ],
)
def _sc_deg(dst3_hbm, z1_hbm, ones_hbm, deg_out, dst2_v, ones_v, zst_v, sem,
            acc):
    c = lax.axis_index("c")
    s = lax.axis_index("s")
    w = c * _NS + s
    pltpu.sync_copy(z1_hbm, zst_v)
    pltpu.sync_copy(ones_hbm, ones_v)
    pltpu.sync_copy(dst3_hbm.at[w], dst2_v)
    for j in range(_NWC):
        pltpu.sync_copy(zst_v, acc.at[pl.ds(s * _TPR + j * _WCH, _WCH)])

    @pl.when(s == 0)
    def _():
        pltpu.sync_copy(zst_v.at[pl.ds(0, _TAIL)],
                        acc.at[pl.ds(_NS * _TPR, _TAIL)])

    plsc.subcore_barrier()

    def fire(i, carry):
        pltpu.async_copy(ones_v, acc.at[dst2_v.at[i]], sem, add=True)
        return carry

    lax.fori_loop(0, _NCHD, fire, 0)

    def drain(i, carry):
        pltpu.make_async_copy(ones_v, acc.at[dst2_v.at[i]], sem).wait()
        return carry

    lax.fori_loop(0, _NCHD, drain, 0)
    plsc.subcore_barrier()
    for j in range(_NWC):
        r0 = s * _TPR + j * _WCH
        pltpu.sync_copy(acc.at[pl.ds(r0, _WCH)], zst_v)
        pltpu.sync_copy(zst_v, deg_out.at[c, pl.ds(r0, _WCH)])

    @pl.when(s == 0)
    def _():
        pltpu.sync_copy(acc.at[pl.ds(_NS * _TPR, _TAIL)],
                        zst_v.at[pl.ds(0, _TAIL)])
        pltpu.sync_copy(zst_v.at[pl.ds(0, _TAIL)],
                        deg_out.at[c, pl.ds(_NS * _TPR, _TAIL)])


@functools.partial(
    pl.kernel,
    mesh=_mesh(),
    compiler_params=_SC_PARAMS,
    out_type=[jax.ShapeDtypeStruct((_NC, _N, _FH), jnp.float32),
              jax.ShapeDtypeStruct((_NS, _NCH, _CH), jnp.float32)],
    scratch_types=[
        pltpu.VMEM((_NCH, _CH), jnp.int32),
        pltpu.VMEM((_NCH, _CH), jnp.int32),
        pltpu.VMEM((_CH, _FH), jnp.float32),
        pltpu.VMEM((_CH, _FH), jnp.float32),
        pltpu.VMEM((_CH, _FH), jnp.float32),
        pltpu.VMEM((_WCHF, _FH), jnp.float32),
        pltpu.SemaphoreType.DMA,
        pltpu.SemaphoreType.DMA,
        pltpu.SemaphoreType.DMA,
        pltpu.SemaphoreType.DMA,
        pltpu.SemaphoreType.DMA,
        pltpu.SemaphoreType.DMA,
        pltpu.VMEM((_NA,), jnp.float32),
        pltpu.VMEM((_NCH, _CH), jnp.float32),
        pltpu.VMEM_SHARED((_NA, _FH), jnp.float32),
    ],
)
def _sc_agg(src3, dst3, x_hbm, z_hbm, svec_hbm, out_hbm, eout_hbm,
            src2, dst2, rb0, rb1, rb2, stage, sg0, sg1, sg2,
            ss0, ss1, ss2, svec_v, ebuf, acc):
    c = lax.axis_index("c")
    s = lax.axis_index("s")
    rows = [rb0, rb1, rb2]
    gsem = [sg0, sg1, sg2]
    ssem = [ss0, ss1, ss2]

    pltpu.sync_copy(src3.at[s], src2)
    pltpu.sync_copy(dst3.at[s], dst2)
    xh = x_hbm.at[c]
    pltpu.async_copy(xh.at[src2.at[0]], rows[0], gsem[0])

    pltpu.sync_copy(z_hbm.at[pl.ds(0, _WCHF)], stage)
    for j in range(_NWCF):
        pltpu.sync_copy(stage, acc.at[pl.ds(s * _TPR + j * _WCHF, _WCHF)])

    @pl.when(s == 0)
    def _():
        pltpu.sync_copy(stage.at[pl.ds(0, _TAIL)],
                        acc.at[pl.ds(_NS * _TPR, _TAIL)])

    @pl.when(c == 0)
    def _():
        pltpu.sync_copy(svec_hbm, svec_v)

    plsc.subcore_barrier()

    def edge_scalars(ci, er):
        for j2 in range(_CH // 16):
            si = src2[ci, pl.ds(j2 * 16, 16)]
            di = dst2[ci, pl.ds(j2 * 16, 16)]
            gs = plsc.load_gather(svec_v, [si])
            gd = plsc.load_gather(svec_v, [di])
            ebuf[er, pl.ds(j2 * 16, 16)] = gs * gd

    def chunk_step(ci, j):
        bp = (j + 1) % 3

        @pl.when(ci >= 2)
        def _():
            pltpu.make_async_copy(rows[bp], acc.at[dst2.at[ci - 2]],
                                  ssem[bp]).wait()

        @pl.when(ci + 1 < _NCH)
        def _():
            pltpu.async_copy(xh.at[src2.at[ci + 1]], rows[bp], gsem[bp])

        pltpu.make_async_copy(xh.at[src2.at[ci]], rows[j], gsem[j]).wait()

        @pl.when(c == 0)
        def _():
            edge_scalars(ci, ci)

        pltpu.async_copy(rows[j], acc.at[dst2.at[ci]], ssem[j], add=True)

    def triple(k, carry):
        for j in range(3):
            chunk_step(3 * k + j, j)
        return carry

    lax.fori_loop(0, _NCH // 3, triple, 0)
    chunk_step(_NCH - 1, 0)
    pltpu.make_async_copy(rows[2], acc.at[dst2.at[_NCH - 2]], ssem[2]).wait()
    pltpu.make_async_copy(rows[0], acc.at[dst2.at[_NCH - 1]], ssem[0]).wait()

    @pl.when(c == 0)
    def _():
        pltpu.sync_copy(ebuf, eout_hbm.at[s])

    plsc.subcore_barrier()
    for j in range(_NWCF):
        r0 = s * _TPR + j * _WCHF
        pltpu.sync_copy(acc.at[pl.ds(r0, _WCHF)], stage)
        pltpu.sync_copy(stage, out_hbm.at[c, pl.ds(r0, _WCHF)])

    @pl.when(s == 0)
    def _():
        pltpu.sync_copy(acc.at[pl.ds(_NS * _TPR, _TAIL)],
                        stage.at[pl.ds(0, _TAIL)])
        pltpu.sync_copy(stage.at[pl.ds(0, _TAIL)],
                        out_hbm.at[c, pl.ds(_NS * _TPR, _TAIL)])


@functools.partial(
    pl.kernel,
    mesh=_mesh(),
    compiler_params=_SC_PARAMS,
    out_type=[jax.ShapeDtypeStruct((_NC, _N, _FH), jnp.float32)],
    scratch_types=[
        pltpu.VMEM((_NCH, _CH), jnp.int32),
        pltpu.VMEM((_NCH, _CH), jnp.int32),
        pltpu.VMEM((_CH, _FH), jnp.float32),
        pltpu.VMEM((_CH, _FH), jnp.float32),
        pltpu.VMEM((_CH, _FH), jnp.float32),
        pltpu.VMEM((_CH, _FH), jnp.float32),
        pltpu.VMEM((_WCH, _FH), jnp.float32),
        pltpu.SemaphoreType.DMA,
        pltpu.SemaphoreType.DMA,
        pltpu.SemaphoreType.DMA,
        pltpu.SemaphoreType.DMA,
        pltpu.SemaphoreType.DMA,
        pltpu.SemaphoreType.DMA,
        pltpu.SemaphoreType.DMA,
        pltpu.SemaphoreType.DMA,
        pltpu.VMEM_SHARED((_NA, _FH), jnp.float32),
    ],
)
def _sc_agg_plain(src3, dst3, x_hbm, z_hbm, out_hbm,
                  src2, dst2, rb0, rb1, rb2, rb3, stage,
                  sg0, sg1, sg2, sg3, ss0, ss1, ss2, ss3, acc):
    c = lax.axis_index("c")
    s = lax.axis_index("s")
    rows = [rb0, rb1, rb2, rb3]
    gsem = [sg0, sg1, sg2, sg3]
    ssem = [ss0, ss1, ss2, ss3]

    pltpu.sync_copy(src3.at[s], src2)
    pltpu.sync_copy(dst3.at[s], dst2)
    xh = x_hbm.at[c]
    pltpu.async_copy(xh.at[src2.at[0]], rows[0], gsem[0])
    pltpu.async_copy(xh.at[src2.at[1]], rows[1], gsem[1])

    pltpu.sync_copy(z_hbm, stage)
    for j in range(_NWC):
        pltpu.sync_copy(stage, acc.at[pl.ds(s * _TPR + j * _WCH, _WCH)])

    @pl.when(s == 0)
    def _():
        pltpu.sync_copy(stage.at[pl.ds(0, _TAIL)],
                        acc.at[pl.ds(_NS * _TPR, _TAIL)])

    plsc.subcore_barrier()

    def chunk_step(ci, j):
        bp = (j + 2) % 4

        @pl.when(ci >= 2)
        def _():
            pltpu.make_async_copy(rows[bp], acc.at[dst2.at[ci - 2]],
                                  ssem[bp]).wait()

        @pl.when(ci + 2 < _NCH)
        def _():
            pltpu.async_copy(xh.at[src2.at[ci + 2]], rows[bp], gsem[bp])

        pltpu.make_async_copy(xh.at[src2.at[ci]], rows[j], gsem[j]).wait()
        pltpu.async_copy(rows[j], acc.at[dst2.at[ci]], ssem[j], add=True)

    def quad(k, carry):
        for j in range(4):
            chunk_step(4 * k + j, j)
        return carry

    lax.fori_loop(0, _NCH // 4, quad, 0)
    chunk_step(_NCH - 2, 0)
    chunk_step(_NCH - 1, 1)
    pltpu.make_async_copy(rows[0], acc.at[dst2.at[_NCH - 2]], ssem[0]).wait()
    pltpu.make_async_copy(rows[1], acc.at[dst2.at[_NCH - 1]], ssem[1]).wait()

    plsc.subcore_barrier()
    for j in range(_NWC):
        r0 = s * _TPR + j * _WCH
        pltpu.sync_copy(acc.at[pl.ds(r0, _WCH)], stage)
        pltpu.sync_copy(stage, out_hbm.at[c, pl.ds(r0, _WCH)])

    @pl.when(s == 0)
    def _():
        pltpu.sync_copy(acc.at[pl.ds(_NS * _TPR, _TAIL)],
                        stage.at[pl.ds(0, _TAIL)])
        pltpu.sync_copy(stage.at[pl.ds(0, _TAIL)],
                        out_hbm.at[c, pl.ds(_NS * _TPR, _TAIL)])


def _row_spec(width):
    return pl.BlockSpec((_BLK, width), lambda i: (i, 0))


def _split_spec():
    return pl.BlockSpec((_NC, _BLK, _FH), lambda i: (0, i, 0))


def _full_spec(shape):
    nd = len(shape)
    return pl.BlockSpec(shape, lambda i: (0,) * nd)


def _split_store(ref, y):
    ref[0] = y[:, :_FH]
    ref[1] = y[:, _FH:]


def _split_load(ref):
    return jnp.concatenate([ref[0], ref[1]], axis=-1)


def _tc_scale_body(dp_ref, x_ref, dis_ref, xs_ref):
    deg = dp_ref[0, :, :1] + dp_ref[1, :, :1]
    dis = lax.rsqrt(jnp.maximum(deg, 1.0))
    dis_ref[...] = dis
    _split_store(xs_ref, dis * x_ref[...])


def _tc_scale(degp, x):
    return pl.pallas_call(
        _tc_scale_body,
        grid=(_NBLK,),
        in_specs=[pl.BlockSpec((_NC, _BLK, _DW), lambda i: (0, i, 0)),
                  _row_spec(_F)],
        out_specs=[_row_spec(1), _split_spec()],
        out_shape=[jax.ShapeDtypeStruct((_N, 1), jnp.float32),
                   jax.ShapeDtypeStruct((_NC, _N, _FH), jnp.float32)],
    )(degp, x)


def _tc_layer_body(p_ref, s_ref, w_ref, y_ref):
    p = _split_load(p_ref)
    sv = s_ref[...]
    h = jnp.maximum(jnp.dot(sv * p, w_ref[...],
                            preferred_element_type=jnp.float32), 0.0)
    _split_store(y_ref, sv * h)


def _tc_layer(part, sv, w):
    return pl.pallas_call(
        _tc_layer_body,
        grid=(_NBLK,),
        in_specs=[_split_spec(), _row_spec(1), _full_spec((_F, _F))],
        out_specs=_split_spec(),
        out_shape=jax.ShapeDtypeStruct((_NC, _N, _FH), jnp.float32),
    )(part, sv, w)


def _tc_att_body(p_ref, dis_ref, w2_ref, wa_ref, ba_ref, nz_ref, tr_ref,
                 x_ref, att_ref, q_ref, xq_ref, info_ref, acc_ref):
    i = pl.program_id(0)
    p = _split_load(p_ref)
    dis = dis_ref[...]
    emb = jnp.maximum(jnp.dot(dis * p, w2_ref[...],
                              preferred_element_type=jnp.float32), 0.0)
    logits = jnp.dot(emb, wa_ref[...],
                     preferred_element_type=jnp.float32) + ba_ref[...]
    att = jax.nn.sigmoid(logits + jnp.where(tr_ref[...] != 0.0,
                                            nz_ref[...], 0.0))
    att_ref[...] = att
    q = dis * att
    q_ref[...] = q
    _split_store(xq_ref, q * x_ref[...])
    r = 0.7
    f = (att * jnp.log(att / r + 1e-6)
         + (1.0 - att) * jnp.log((1.0 - att) / (1.0 - r + 1e-6) + 1e-6))
    part = jnp.sum(f).reshape(1, 1)
    acc_ref[...] = jnp.where(i == 0, part, acc_ref[...] + part)

    @pl.when(i == _NBLK - 1)
    def _():
        info_ref[...] = acc_ref[...] / float(_N)


def _tc_att(part, dis, w2, wa, ba, noise, tr, x):
    return pl.pallas_call(
        _tc_att_body,
        grid=(_NBLK,),
        in_specs=[_split_spec(), _row_spec(1), _full_spec((_F, _F)),
                  _full_spec((_F, 1)), _full_spec((1, 1)), _row_spec(1),
                  _full_spec((1, 1)), _row_spec(_F)],
        out_specs=[_row_spec(1), _row_spec(1), _split_spec(),
                   _full_spec((1, 1))],
        out_shape=[jax.ShapeDtypeStruct((_N, 1), jnp.float32),
                   jax.ShapeDtypeStruct((_N, 1), jnp.float32),
                   jax.ShapeDtypeStruct((_NC, _N, _FH), jnp.float32),
                   jax.ShapeDtypeStruct((1, 1), jnp.float32)],
        scratch_shapes=[pltpu.VMEM((1, 1), jnp.float32)],
    )(part, dis, w2, wa, ba, noise, tr, x)


def _tc_pool_body(p_ref, q_ref, w4_ref, b_ref, sp_ref, pool_ref, cnt_ref):
    i = pl.program_id(0)
    p = _split_load(p_ref)
    s2 = jnp.maximum(q_ref[...] * jnp.dot(p, w4_ref[...],
                                          preferred_element_type=jnp.float32),
                     0.0)
    b = b_ref[...]
    iota = lax.broadcasted_iota(jnp.int32, (_BLK, _G), 1)
    m = (b == iota).astype(jnp.float32)
    dims = (((0,), (0,)), ((), ()))
    pool_d = lax.dot_general(m, s2, dims, preferred_element_type=jnp.float32)
    cnt_d = lax.dot_general(m, jnp.ones((_BLK, 1), jnp.float32), dims,
                            preferred_element_type=jnp.float32)

    @pl.when(i == 0)
    def _():
        pool_ref[...] = pool_d
        cnt_ref[...] = cnt_d

    @pl.when(i > 0)
    def _():
        pool_ref[...] += pool_d
        cnt_ref[...] += cnt_d

    @pl.when(i == _NBLK - 1)
    def _():
        sp_ref[...] = pool_ref[...] / jnp.maximum(cnt_ref[...], 1.0)


def _tc_pool(part, q, w4, batch2d):
    return pl.pallas_call(
        _tc_pool_body,
        grid=(_NBLK,),
        in_specs=[_split_spec(), _row_spec(1), _full_spec((_F, _F)),
                  _row_spec(1)],
        out_specs=_full_spec((_G, _F)),
        out_shape=jax.ShapeDtypeStruct((_G, _F), jnp.float32),
        scratch_shapes=[pltpu.VMEM((_G, _F), jnp.float32),
                        pltpu.VMEM((_G, 1), jnp.float32)],
    )(part, q, w4, batch2d)


def kernel(edge_index, inputs, epoch, training, batch, W1, W2, Wa, ba, W3, W4):
    pad = _EPTP - _EPT
    if pad:
        src3 = jnp.pad(edge_index[0].reshape(_NS, _EPT),
                       ((0, 0), (0, pad))).reshape(_NS, _NCH, _CH)
        padv = _TRASH + (jnp.arange(pad, dtype=jnp.int32) % _NTR)
        dst3 = jnp.concatenate(
            [edge_index[1].reshape(_NS, _EPT),
             jnp.broadcast_to(padv, (_NS, pad))],
            axis=1).reshape(_NS, _NCH, _CH)
    else:
        src3 = edge_index[0].reshape(_NS, _NCH, _CH)
        dst3 = edge_index[1].reshape(_NS, _NCH, _CH)
    dst3d = edge_index[1].reshape(_NW, _NCHD, _CHD)
    z2 = jnp.zeros((_WCH, _FH), jnp.float32)
    z1 = jnp.zeros((_WCH, _DW), jnp.float32)
    ones1 = jnp.ones((_CHD, _DW), jnp.float32)
    tr = jnp.asarray(training, jnp.float32).reshape(1, 1)
    ba2 = ba.reshape(1, 1)

    u = jax.random.uniform(jax.random.key(42), (_N, 1),
                           minval=1e-10, maxval=1.0 - 1e-10)
    noise = jnp.log(u) - jnp.log(1.0 - u)

    (degp,) = _sc_deg(dst3d, z1, ones1)
    dis, xs = _tc_scale(degp, inputs)
    disf = dis.reshape(_N)

    a1, norm3 = _sc_agg(src3, dst3, xs, z2, disf)
    h1s = _tc_layer(a1, dis, W1)
    (a2,) = _sc_agg_plain(src3, dst3, h1s, z2)
    att, q, xq, info = _tc_att(a2, dis, W2, Wa, ba2, noise, tr, inputs)
    attf = att.reshape(_N)
    a3, eatt3 = _sc_agg(src3, dst3, xq, z2, attf)
    s1q = _tc_layer(a3, q, W3)
    (a4,) = _sc_agg_plain(src3, dst3, s1q, z2)
    sp_emb = _tc_pool(a4, q, W4, batch.reshape(_N, 1))

    edge_att = eatt3.reshape(_NS, _EPTP)[:, :_EPT].reshape(_E, 1)
    edge_weights = norm3.reshape(_NS, _EPTP)[:, :_EPT].reshape(_E)
    info_loss = info[0, 0]
    feat_weights = jnp.ones((_F,), jnp.float32)
    return edge_att, info_loss, sp_emb, edge_weights, feat_weights
